# Initial kernel scaffold; baseline (speedup 1.0000x reference)
import jax
import jax.numpy as jnp
from jax.experimental import pallas as pl


def kernel(pair_states, macro_state, W1, b1, ln1_g, ln1_b, W2, b2, po_W, po_b, po_g, po_beta, pair_mem_keys, pair_mem_vals, macro_mem_keys, macro_mem_vals):
    raise NotImplementedError("write your pallas kernel here")


# speedup vs baseline: 1.0781x; 1.0781x over previous
import functools

import jax
import jax.numpy as jnp
from jax import lax
from jax.experimental import pallas as pl
from jax.experimental.pallas import tpu as pltpu

B = 1024
P = 32
DP = 64
DM = 128
S = 4096
V = 2048

BT = 128
NBT = B // BT
ST = 512
NST = S // ST

_F32 = jnp.float32
_BF16 = jnp.bfloat16
_HI = lax.Precision.HIGHEST


def _ln(x, g, b, eps=1e-5):
    m = jnp.mean(x, axis=-1, keepdims=True)
    v = jnp.mean((x - m) ** 2, axis=-1, keepdims=True)
    return (x - m) / jnp.sqrt(v + eps) * g + b


def _k1_body(act_ref, ms_ref, kp_ref, km_ref,
             attnp_ref, attnm_ref, q_ref, wp_ref, wm_ref, idxp_ref, idxm_ref):
    act = act_ref[...]
    q = act[:, 0:DP]
    for p in range(1, P):
        q = q + act[:, p * DP:(p + 1) * DP]
    q = q * (1.0 / P)
    q_ref[...] = q

    def head(query, keys, scale, attn_ref, w_ref, idx_ref):
        s = lax.dot_general(query.astype(_BF16), keys.astype(_BF16),
                            (((1,), (1,)), ((), ())),
                            preferred_element_type=_F32)
        s = s * scale
        m = jnp.max(s, axis=1, keepdims=True)
        e = jnp.exp(s - m)
        denom = jnp.sum(e, axis=1, keepdims=True)
        attn = e / denom
        attn_ref[...] = attn.astype(_BF16)
        amax = 1.0 / denom
        surprise = 1.0 - amax
        w = 0.1 * jax.nn.sigmoid(surprise)
        w_ref[...] = w.reshape(1, 1, BT)
        ii = lax.broadcasted_iota(jnp.int32, (BT, S), 1)
        sel = jnp.where(s == m, ii, jnp.int32(2**30))
        idx = jnp.min(sel, axis=1)
        idx_ref[...] = idx.reshape(1, 1, BT)

    head(q, kp_ref[...], 1.0 / (DP ** 0.5), attnp_ref, wp_ref, idxp_ref)
    head(ms_ref[...], km_ref[...], 1.0 / (DM ** 0.5), attnm_ref, wm_ref, idxm_ref)


def _k1(actual, macro_state, kp, km):
    out_shape = [
        jax.ShapeDtypeStruct((B, S), _BF16),
        jax.ShapeDtypeStruct((B, S), _BF16),
        jax.ShapeDtypeStruct((B, DP), _F32),
        jax.ShapeDtypeStruct((NBT, 1, BT), _F32),
        jax.ShapeDtypeStruct((NBT, 1, BT), _F32),
        jax.ShapeDtypeStruct((NBT, 1, BT), jnp.int32),
        jax.ShapeDtypeStruct((NBT, 1, BT), jnp.int32),
    ]
    return pl.pallas_call(
        _k1_body,
        grid=(NBT,),
        in_specs=[
            pl.BlockSpec((BT, V), lambda i: (i, 0)),
            pl.BlockSpec((BT, DM), lambda i: (i, 0)),
            pl.BlockSpec((S, DP), lambda i: (0, 0)),
            pl.BlockSpec((S, DM), lambda i: (0, 0)),
        ],
        out_specs=[
            pl.BlockSpec((BT, S), lambda i: (i, 0)),
            pl.BlockSpec((BT, S), lambda i: (i, 0)),
            pl.BlockSpec((BT, DP), lambda i: (i, 0)),
            pl.BlockSpec((1, 1, BT), lambda i: (i, 0, 0)),
            pl.BlockSpec((1, 1, BT), lambda i: (i, 0, 0)),
            pl.BlockSpec((1, 1, BT), lambda i: (i, 0, 0)),
            pl.BlockSpec((1, 1, BT), lambda i: (i, 0, 0)),
        ],
        out_shape=out_shape,
    )(actual, macro_state, kp, km)


def _mm_body(a_ref, b_ref, o_ref):
    o_ref[...] = jnp.dot(a_ref[...], b_ref[...],
                         preferred_element_type=_F32).astype(_BF16)


def _k2(attn, vals, n):
    return pl.pallas_call(
        _mm_body,
        grid=(NBT,),
        in_specs=[
            pl.BlockSpec((BT, S), lambda i: (i, 0)),
            pl.BlockSpec((S, n), lambda i: (0, 0)),
        ],
        out_specs=pl.BlockSpec((BT, n), lambda i: (i, 0)),
        out_shape=jax.ShapeDtypeStruct((B, n), _BF16),
    )(attn, vals)


def _k3_body(rp_ref, rm_ref, w1a_ref, w1b_ref, b1_ref, g_ref, be_ref, o_ref):
    h = jnp.dot(rp_ref[...], w1a_ref[...], preferred_element_type=_F32)
    h = h + jnp.dot(rm_ref[...], w1b_ref[...], preferred_element_type=_F32)
    h = h + b1_ref[...]
    h = _ln(h, g_ref[...], be_ref[...])
    h = 0.5 * h * (1.0 + lax.erf(h * (2.0 ** -0.5)))
    o_ref[...] = h.astype(_BF16)


def _k3(rp, rm, w1a, w1b, b1, g, be):
    return pl.pallas_call(
        _k3_body,
        grid=(NBT,),
        in_specs=[
            pl.BlockSpec((BT, V), lambda i: (i, 0)),
            pl.BlockSpec((BT, V), lambda i: (i, 0)),
            pl.BlockSpec((V, V), lambda i: (0, 0)),
            pl.BlockSpec((V, V), lambda i: (0, 0)),
            pl.BlockSpec((1, V), lambda i: (0, 0)),
            pl.BlockSpec((1, V), lambda i: (0, 0)),
            pl.BlockSpec((1, V), lambda i: (0, 0)),
        ],
        out_specs=pl.BlockSpec((BT, V), lambda i: (i, 0)),
        out_shape=jax.ShapeDtypeStruct((B, V), _BF16),
    )(rp, rm, w1a, w1b, b1, g, be)


def _k4_body(h_ref, w2_ref, b2_ref, ps_ref, pow_ref, pob_ref, pog_ref,
             pobe_ref, o_ref):
    fused = jnp.dot(h_ref[...], w2_ref[...], preferred_element_type=_F32)
    fused = fused + b2_ref[...]
    for p in range(P):
        xp = jnp.concatenate(
            [ps_ref[:, p, :].astype(_BF16),
             fused[:, p * DP:(p + 1) * DP].astype(_BF16)], axis=1)
        e = jnp.dot(xp, pow_ref[p], preferred_element_type=_F32)
        e = e + pob_ref[p:p + 1, :]
        e = _ln(e, pog_ref[p:p + 1, :], pobe_ref[p:p + 1, :])
        o_ref[:, p, :] = e


def _k4(h, w2, b2, ps, po_w, po_b, po_g, po_beta):
    return pl.pallas_call(
        _k4_body,
        grid=(NBT,),
        in_specs=[
            pl.BlockSpec((BT, V), lambda i: (i, 0)),
            pl.BlockSpec((V, V), lambda i: (0, 0)),
            pl.BlockSpec((1, V), lambda i: (0, 0)),
            pl.BlockSpec((BT, P, DP), lambda i: (i, 0, 0)),
            pl.BlockSpec((P, 2 * DP, DP), lambda i: (0, 0, 0)),
            pl.BlockSpec((P, DP), lambda i: (0, 0)),
            pl.BlockSpec((P, DP), lambda i: (0, 0)),
            pl.BlockSpec((P, DP), lambda i: (0, 0)),
        ],
        out_specs=pl.BlockSpec((BT, P, DP), lambda i: (i, 0, 0)),
        out_shape=jax.ShapeDtypeStruct((B, P, DP), _F32),
    )(h, w2, b2, ps, po_w, po_b, po_g, po_beta)


def _k5_body(idxp_ref, idxm_ref, wp_ref, wm_ref, act_ref, q_ref, ms_ref,
             pk_ref, pv_ref, mk_ref, mv_ref,
             npk_ref, npv_ref, nmk_ref, nmv_ref,
             accpv, accmv, accpk, accmk, dp_acc, dm_acc):
    sblk = pl.program_id(0)
    b = pl.program_id(1)

    rows = sblk * ST + lax.broadcasted_iota(jnp.int32, (ST, BT), 0)

    def accum(idx_ref, w_ref, one_hot_out):
        idx = idx_ref[0]
        w = w_ref[0]
        hit = rows == jnp.broadcast_to(idx, (ST, BT))
        a = jnp.where(hit, jnp.broadcast_to(w, (ST, BT)), 0.0)
        return a

    ap = accum(idxp_ref, wp_ref, None)
    am = accum(idxm_ref, wm_ref, None)
    act = act_ref[...]
    qb = q_ref[...]
    msb = ms_ref[...]

    cpv = jnp.dot(ap.astype(_BF16), act, preferred_element_type=_F32)
    cmv = jnp.dot(am.astype(_BF16), act, preferred_element_type=_F32)
    cpk = jnp.dot(ap.astype(_BF16), qb, preferred_element_type=_F32)
    cmk = jnp.dot(am.astype(_BF16), msb, preferred_element_type=_F32)
    cdp = jnp.sum(ap, axis=1, keepdims=True)
    cdm = jnp.sum(am, axis=1, keepdims=True)

    @pl.when(b == 0)
    def _():
        accpv[...] = cpv
        accmv[...] = cmv
        accpk[...] = cpk
        accmk[...] = cmk
        dp_acc[...] = cdp
        dm_acc[...] = cdm

    @pl.when(b != 0)
    def _():
        accpv[...] += cpv
        accmv[...] += cmv
        accpk[...] += cpk
        accmk[...] += cmk
        dp_acc[...] += cdp
        dm_acc[...] += cdm

    @pl.when(b == NBT - 1)
    def _():
        npv_ref[...] = pv_ref[...] * (1.0 - dp_acc[...]) + accpv[...]
        nmv_ref[...] = mv_ref[...] * (1.0 - dm_acc[...]) + accmv[...]
        npk_ref[...] = pk_ref[...] * (1.0 - dp_acc[...]) + accpk[...]
        nmk_ref[...] = mk_ref[...] * (1.0 - dm_acc[...]) + accmk[...]


def _k5(idxp, idxm, wp, wm, act_bf, q_bf, ms_bf, pk, pv, mk, mv):
    out_shape = [
        jax.ShapeDtypeStruct((S, DP), _F32),
        jax.ShapeDtypeStruct((S, V), _F32),
        jax.ShapeDtypeStruct((S, DM), _F32),
        jax.ShapeDtypeStruct((S, V), _F32),
    ]
    bspec = lambda shape, imap: pl.BlockSpec(shape, imap)
    return pl.pallas_call(
        _k5_body,
        grid=(NST, NBT),
        in_specs=[
            pl.BlockSpec((1, 1, BT), lambda s_, b_: (b_, 0, 0)),
            pl.BlockSpec((1, 1, BT), lambda s_, b_: (b_, 0, 0)),
            pl.BlockSpec((1, 1, BT), lambda s_, b_: (b_, 0, 0)),
            pl.BlockSpec((1, 1, BT), lambda s_, b_: (b_, 0, 0)),
            pl.BlockSpec((BT, V), lambda s_, b_: (b_, 0)),
            pl.BlockSpec((BT, DP), lambda s_, b_: (b_, 0)),
            pl.BlockSpec((BT, DM), lambda s_, b_: (b_, 0)),
            pl.BlockSpec((ST, DP), lambda s_, b_: (s_, 0)),
            pl.BlockSpec((ST, V), lambda s_, b_: (s_, 0)),
            pl.BlockSpec((ST, DM), lambda s_, b_: (s_, 0)),
            pl.BlockSpec((ST, V), lambda s_, b_: (s_, 0)),
        ],
        out_specs=[
            pl.BlockSpec((ST, DP), lambda s_, b_: (s_, 0)),
            pl.BlockSpec((ST, V), lambda s_, b_: (s_, 0)),
            pl.BlockSpec((ST, DM), lambda s_, b_: (s_, 0)),
            pl.BlockSpec((ST, V), lambda s_, b_: (s_, 0)),
        ],
        out_shape=out_shape,
        scratch_shapes=[
            pltpu.VMEM((ST, V), _F32),
            pltpu.VMEM((ST, V), _F32),
            pltpu.VMEM((ST, DP), _F32),
            pltpu.VMEM((ST, DM), _F32),
            pltpu.VMEM((ST, 1), _F32),
            pltpu.VMEM((ST, 1), _F32),
        ],
    )(idxp, idxm, wp, wm, act_bf, q_bf, ms_bf, pk, pv, mk, mv)


def kernel(pair_states, macro_state, W1, b1, ln1_g, ln1_b, W2, b2,
           po_W, po_b, po_g, po_beta,
           pair_mem_keys, pair_mem_vals, macro_mem_keys, macro_mem_vals):
    actual = pair_states.reshape(B, V)

    attn_p, attn_m, q, wp, wm, idxp, idxm = _k1(
        actual, macro_state, pair_mem_keys, macro_mem_keys)

    rp = _k2(attn_p, pair_mem_vals.astype(_BF16), V)
    rm = _k2(attn_m, macro_mem_vals.astype(_BF16), V)

    h = _k3(rp, rm,
            W1[:V].astype(_BF16), W1[V:].astype(_BF16),
            b1.reshape(1, V), ln1_g.reshape(1, V), ln1_b.reshape(1, V))

    enriched = _k4(h, W2.astype(_BF16), b2.reshape(1, V), pair_states,
                   po_W.astype(_BF16), po_b, po_g, po_beta)

    new_pk, new_pv, new_mk, new_mv = _k5(
        idxp, idxm, wp, wm,
        actual.astype(_BF16), q.astype(_BF16), macro_state.astype(_BF16),
        pair_mem_keys, pair_mem_vals, macro_mem_keys, macro_mem_vals)

    return (enriched, new_pk, new_pv, new_mk, new_mv)
